# baseline (speedup 1.0000x reference)
import functools
import jax
import jax.numpy as jnp
from jax import lax
from jax.experimental import pallas as pl
from jax.experimental.pallas import tpu as pltpu
from jax.experimental.pallas import tpu_sc as plsc

_NC = 2
_NS = 16
_NW = _NC * _NS
_B = 16384
_TOT = 2 * _B
_PER_W = _TOT // _NW
_HALF_W = _PER_W // 2
_CHUNK = 128
_NCHUNK = _PER_W // _CHUNK
_D = 64
_MARGIN = 4.0


def _rsqrt16(x):
    i = plsc.bitcast(x, jnp.int32)
    i = 0x5F3759DF - lax.shift_right_logical(i, 1)
    y = plsc.bitcast(i, jnp.float32)
    for _ in range(3):
        y = y * (1.5 - 0.5 * x * y * y)
    return y


def _sc_call(entity_emb, relation_emb, idx_blocks):
    mesh = plsc.VectorSubcoreMesh(
        core_axis_name="c", subcore_axis_name="s",
        num_cores=_NC, num_subcores=_NS)

    @functools.partial(
        pl.kernel,
        out_type=(
            jax.ShapeDtypeStruct((_TOT,), jnp.float32),
            jax.ShapeDtypeStruct((_NW, 16), jnp.float32),
        ),
        mesh=mesh,
        compiler_params=pltpu.CompilerParams(
            needs_layout_passes=False, use_tc_tiling_on_sc=False),
        scratch_types=[
            pltpu.VMEM((5, _CHUNK), jnp.int32),
            pltpu.VMEM((_CHUNK, _D), jnp.float32),
            pltpu.VMEM((_CHUNK, _D), jnp.float32),
            pltpu.VMEM((_CHUNK, _D), jnp.float32),
            pltpu.VMEM((_CHUNK, _D), jnp.float32),
            pltpu.VMEM((_CHUNK, _D), jnp.float32),
            pltpu.VMEM((_PER_W,), jnp.float32),
            pltpu.VMEM((16,), jnp.float32),
            pltpu.SemaphoreType.DMA,
        ],
    )
    def trans_e(ent_hbm, rel_hbm, idx_hbm, scores_hbm, part_hbm,
                idx_v, h_v, r_v, t_v, qr_v, qe_v, sc_v, par_v, sem):
        w = lax.axis_index("s") * _NC + lax.axis_index("c")
        lane = jnp.arange(16, dtype=jnp.int32)
        zero = jnp.zeros((16,), jnp.float32)

        def chunk_body(c, carry):
            pltpu.sync_copy(idx_hbm.at[w, c], idx_v)
            cps = [
                pltpu.async_copy(ent_hbm.at[idx_v.at[0]], h_v, sem),
                pltpu.async_copy(rel_hbm.at[idx_v.at[1]], r_v, sem),
                pltpu.async_copy(ent_hbm.at[idx_v.at[2]], t_v, sem),
                pltpu.async_copy(rel_hbm.at[idx_v.at[3]], qr_v, sem),
                pltpu.async_copy(ent_hbm.at[idx_v.at[4]], qe_v, sem),
            ]
            for cp in cps:
                cp.wait()

            def group_body(g, gcarry):
                rows = g * 16 + lane

                def norm_body(d, acc):
                    sh, st, sq = acc
                    dv = jnp.full((16,), d, jnp.int32)
                    hv = plsc.load_gather(h_v, [rows, dv])
                    tv = plsc.load_gather(t_v, [rows, dv])
                    qv = plsc.load_gather(qe_v, [rows, dv])
                    return (sh + hv * hv, st + tv * tv, sq + qv * qv)

                sh, st, sq = lax.fori_loop(
                    0, _D, norm_body, (zero, zero, zero), unroll=8)
                inv_h = _rsqrt16(sh)
                inv_t = _rsqrt16(st)
                inv_q = _rsqrt16(sq)

                def comb_body(d, acc):
                    dv = jnp.full((16,), d, jnp.int32)
                    hv = plsc.load_gather(h_v, [rows, dv])
                    rv = plsc.load_gather(r_v, [rows, dv])
                    tv = plsc.load_gather(t_v, [rows, dv])
                    qrv = plsc.load_gather(qr_v, [rows, dv])
                    qev = plsc.load_gather(qe_v, [rows, dv])
                    s = hv * inv_h + rv - tv * inv_t + qrv - qev * inv_q
                    return acc + jnp.abs(s)

                dist = lax.fori_loop(0, _D, comb_body, zero, unroll=8)
                sc_v[pl.ds(c * _CHUNK + g * 16, 16)] = dist
                return gcarry

            lax.fori_loop(0, _CHUNK // 16, group_body, 0)
            return carry

        lax.fori_loop(0, _NCHUNK, chunk_body, 0)

        def loss_body(i, p):
            pv = sc_v[pl.ds(i * 16, 16)]
            nv = sc_v[pl.ds(_HALF_W + i * 16, 16)]
            return p + jnp.maximum(pv - nv + _MARGIN, 0.0)

        par_v[...] = lax.fori_loop(0, _HALF_W // 16, loss_body, zero,
                                   unroll=4)
        pltpu.sync_copy(sc_v.at[pl.ds(0, _HALF_W)],
                        scores_hbm.at[pl.ds(w * _HALF_W, _HALF_W)])
        pltpu.sync_copy(sc_v.at[pl.ds(_HALF_W, _HALF_W)],
                        scores_hbm.at[pl.ds(_B + w * _HALF_W, _HALF_W)])
        pltpu.sync_copy(par_v, part_hbm.at[w])

    return trans_e(entity_emb, relation_emb, idx_blocks)


def kernel(entity_emb, relation_emb, batch_positives, batch_negatives):
    pos = batch_positives.reshape(_NW, _NCHUNK // 2, _CHUNK, 5)
    neg = batch_negatives.reshape(_NW, _NCHUNK // 2, _CHUNK, 5)
    idx_blocks = jnp.concatenate(
        [pos.transpose(0, 1, 3, 2), neg.transpose(0, 1, 3, 2)], axis=1)
    scores, partials = _sc_call(entity_emb, relation_emb, idx_blocks)
    loss = jnp.sum(partials) / _B
    return scores[:_B], scores[_B:], loss
